# baseline (speedup 1.0000x reference)
import functools

import jax
import jax.numpy as jnp
from jax import lax
from jax.experimental import pallas as pl
from jax.experimental.pallas import tpu as pltpu
from jax.experimental.pallas import tpu_sc as plsc

VOCAB = 1000000
DIM = 64
B = 16384
CTX = 5

NC = 2
NS = 16
L = 16
NW = NC * NS
BPW = B // NW
CB = 128
NCHUNK = BPW // CB
PAIRS = CB * CTX
DW = DIM // 2


def _skipgram_body(tgt_hbm, ctx_hbm, tt_hbm, ct_hbm, out_hbm,
                   ti_v, craw_v, ci_v, we_i, ce_i, out_v, sem):
    wid = lax.axis_index("s") * NC + lax.axis_index("c")
    base = wid * BPW

    lanes = lax.iota(jnp.int32, L)

    for g in range(NCHUNK):
        b0 = base + g * CB
        pltpu.sync_copy(tgt_hbm.at[pl.ds(b0, CB)], ti_v)
        pltpu.sync_copy(ctx_hbm.at[pl.ds(b0 * CTX, PAIRS)], craw_v)

        def regroup(i, _):
            b16 = i * L + lanes
            for c in range(CTX):
                vals = plsc.load_gather(craw_v, [b16 * CTX + c])
                plsc.store_scatter(ci_v, [jnp.full((L,), c, jnp.int32), b16], vals)
            return 0

        lax.fori_loop(0, CB // L, regroup, 0)

        cps = [pltpu.async_copy(tt_hbm.at[ti_v], we_i, sem)]
        for c in range(CTX):
            cps.append(pltpu.async_copy(ct_hbm.at[ci_v.at[c]], ce_i.at[c], sem))
        for cp in cps:
            cp.wait()

        def group(bg, _):
            b16 = bg * L + lanes
            czero = [jnp.full((L,), c, jnp.int32) for c in range(CTX)]

            def dstep(d, accs):
                ds = jnp.full((L,), 0, jnp.int32) + d
                wlo, whi = plsc.unpack(
                    plsc.bitcast(plsc.load_gather(we_i, [b16, ds]), jnp.bfloat16),
                    format=plsc.PackFormat.INTERLEAVED,
                    preferred_element_type=jnp.float32)
                new = []
                for c in range(CTX):
                    clo, chi = plsc.unpack(
                        plsc.bitcast(plsc.load_gather(ce_i, [czero[c], b16, ds]),
                                     jnp.bfloat16),
                        format=plsc.PackFormat.INTERLEAVED,
                        preferred_element_type=jnp.float32)
                    new.append(accs[c] + clo * wlo + chi * whi)
                return tuple(new)

            accs = lax.fori_loop(
                0, DW, dstep,
                tuple(jnp.zeros((L,), jnp.float32) for _ in range(CTX)),
                unroll=4)
            for c in range(CTX):
                plsc.store_scatter(out_v, [b16 * CTX + c], accs[c])
            return 0

        lax.fori_loop(0, CB // L, group, 0)
        pltpu.sync_copy(out_v, out_hbm.at[pl.ds(b0 * CTX, PAIRS)])


def kernel(target, context, target_table, context_table):
    tgt = target.reshape(B)
    ctx = context.reshape(B * CTX)
    tt16 = jax.lax.bitcast_convert_type(
        target_table.astype(jnp.bfloat16).reshape(VOCAB, DW, 2), jnp.int32)
    ct16 = jax.lax.bitcast_convert_type(
        context_table.astype(jnp.bfloat16).reshape(VOCAB, DW, 2), jnp.int32)

    run = pl.kernel(
        _skipgram_body,
        out_type=jax.ShapeDtypeStruct((B * CTX,), jnp.float32),
        mesh=plsc.VectorSubcoreMesh(core_axis_name="c", subcore_axis_name="s"),
        scratch_types=[
            pltpu.VMEM((CB,), jnp.int32),
            pltpu.VMEM((PAIRS,), jnp.int32),
            pltpu.VMEM((CTX, CB), jnp.int32),
            pltpu.VMEM((CB, DW), jnp.int32),
            pltpu.VMEM((CTX, CB, DW), jnp.int32),
            pltpu.VMEM((PAIRS,), jnp.float32),
            pltpu.SemaphoreType.DMA,
        ],
        compiler_params=pltpu.CompilerParams(
            needs_layout_passes=False, use_tc_tiling_on_sc=False),
    )
    out = run(tgt, ctx, tt16, ct16)
    return out.reshape(B, CTX)


# speedup vs baseline: 2.7283x; 2.7283x over previous
import functools

import jax
import jax.numpy as jnp
from jax import lax
from jax.experimental import pallas as pl
from jax.experimental.pallas import tpu as pltpu
from jax.experimental.pallas import tpu_sc as plsc

VOCAB = 1000000
DIM = 64
B = 16384
CTX = 5

NC = 2
NS = 16
L = 16
NW = NC * NS
BPW = B // NW
CB = 128
NCHUNK = BPW // CB
PAIRS = CB * CTX
NBUF = 2


def _skipgram_body(tgt_hbm, ctx_hbm, tt_hbm, ct_hbm, out_hbm,
                   ti_v, craw_v, ci_v, we_v, ce_v, out_v, sems):
    wid = lax.axis_index("s") * NC + lax.axis_index("c")
    base = wid * BPW

    lanes = lax.iota(jnp.int32, L)

    def stage_and_fire(g, s):
        b0 = base + g * CB
        pltpu.sync_copy(tgt_hbm.at[pl.ds(b0, CB)], ti_v.at[s])
        pltpu.sync_copy(ctx_hbm.at[pl.ds(b0 * CTX, PAIRS)], craw_v.at[s])

        def regroup(i, _):
            b16 = i * L + lanes
            for c in range(CTX):
                vals = plsc.load_gather(craw_v, [jnp.full((L,), s, jnp.int32),
                                                 b16 * CTX + c])
                plsc.store_scatter(
                    ci_v, [jnp.full((L,), s, jnp.int32),
                           jnp.full((L,), c, jnp.int32), b16], vals)
            return 0

        lax.fori_loop(0, CB // L, regroup, 0)

        cps = [pltpu.async_copy(tt_hbm.at[ti_v.at[s]], we_v.at[s], sems[s])]
        for c in range(CTX):
            cps.append(pltpu.async_copy(
                ct_hbm.at[ci_v.at[s, c]], ce_v.at[s, c], sems[s]))
        return cps

    def compute(g, s, cps):
        b0 = base + g * CB
        for cp in cps:
            cp.wait()

        def group(bg, _):
            b16 = bg * L + lanes
            szero = jnp.full((L,), s, jnp.int32)
            czero = [jnp.full((L,), c, jnp.int32) for c in range(CTX)]

            def dstep(d, accs):
                ds = jnp.full((L,), 0, jnp.int32) + d
                wv = plsc.load_gather(we_v, [szero, b16, ds])
                return tuple(
                    accs[c] + plsc.load_gather(ce_v, [szero, czero[c], b16, ds]) * wv
                    for c in range(CTX)
                )

            accs = lax.fori_loop(
                0, DIM, dstep,
                tuple(jnp.zeros((L,), jnp.float32) for _ in range(CTX)),
                unroll=4)
            for c in range(CTX):
                plsc.store_scatter(out_v, [b16 * CTX + c], accs[c])
            return 0

        lax.fori_loop(0, CB // L, group, 0)
        pltpu.sync_copy(out_v, out_hbm.at[pl.ds(b0 * CTX, PAIRS)])

    cps = stage_and_fire(0, 0)
    for g in range(NCHUNK):
        nxt = stage_and_fire(g + 1, (g + 1) % NBUF) if g + 1 < NCHUNK else None
        compute(g, g % NBUF, cps)
        cps = nxt


def kernel(target, context, target_table, context_table):
    tgt = target.reshape(B)
    ctx = context.reshape(B * CTX)

    run = pl.kernel(
        _skipgram_body,
        out_type=jax.ShapeDtypeStruct((B * CTX,), jnp.float32),
        mesh=plsc.VectorSubcoreMesh(core_axis_name="c", subcore_axis_name="s"),
        scratch_types=[
            pltpu.VMEM((NBUF, CB), jnp.int32),
            pltpu.VMEM((NBUF, PAIRS), jnp.int32),
            pltpu.VMEM((NBUF, CTX, CB), jnp.int32),
            pltpu.VMEM((NBUF, CB, DIM), jnp.float32),
            pltpu.VMEM((NBUF, CTX, CB, DIM), jnp.float32),
            pltpu.VMEM((PAIRS,), jnp.float32),
            [pltpu.SemaphoreType.DMA] * NBUF,
        ],
        compiler_params=pltpu.CompilerParams(
            needs_layout_passes=False, use_tc_tiling_on_sc=False),
    )
    out = run(tgt, ctx, target_table, context_table)
    return out.reshape(B, CTX)
